# baseline (speedup 1.0000x reference)
import jax
import jax.numpy as jnp
from jax.experimental import pallas as pl

_BB = 256


def _add_kernel(x_ref, e_ref, o_ref):
    ld = x_ref.shape[1]
    o_ref[...] = x_ref[...] + e_ref[:, :ld]


def kernel(x, encoding):
    B, L, D = x.shape
    xf = x.reshape(B, L * D)
    ef = encoding.reshape(1, encoding.shape[0] * D)
    grid = (B // _BB,)
    out = pl.pallas_call(
        _add_kernel,
        grid=grid,
        in_specs=[
            pl.BlockSpec((_BB, L * D), lambda i: (i, 0)),
            pl.BlockSpec((1, ef.shape[1]), lambda i: (0, 0)),
        ],
        out_specs=pl.BlockSpec((_BB, L * D), lambda i: (i, 0)),
        out_shape=jax.ShapeDtypeStruct((B, L * D), x.dtype),
    )(xf, ef)
    return out.reshape(B, L, D)
